# initial kernel scaffold (unmeasured)
import jax
import jax.numpy as jnp
from jax import lax
from jax.experimental import pallas as pl
from jax.experimental.pallas import tpu as pltpu

N_DEV = 4
MAX_ROWS = 544


def kernel(x, dest):
    S, D = x.shape

    order = jnp.argsort(dest, stable=True)
    x_sorted = jnp.take(x, order, axis=0)
    counts = jnp.zeros((N_DEV,), jnp.int32).at[dest].add(1)
    offs = jnp.concatenate(
        [jnp.zeros((1,), jnp.int32), jnp.cumsum(counts)[:-1].astype(jnp.int32)]
    )
    x_pad = jnp.pad(x_sorted, ((0, MAX_ROWS), (0, 0)))
    counts_pad = jnp.zeros((8, 128), jnp.int32).at[0, :N_DEV].set(counts)

    def body(offs_ref, x_ref, counts_ref, stage_ref, cstage_ref,
             send_sems, recv_sems, csend_sems, crecv_sems):
        my = lax.axis_index("i")

        barrier_sem = pltpu.get_barrier_semaphore()
        for dr in range(1, N_DEV):
            peer = lax.rem(my + dr, N_DEV)
            pl.semaphore_signal(
                barrier_sem, inc=1,
                device_id=(peer,), device_id_type=pl.DeviceIdType.MESH,
            )
        pl.semaphore_wait(barrier_sem, N_DEV - 1)

        rdmas = []
        for dr in range(1, N_DEV):
            r = lax.rem(my + dr, N_DEV)
            data = pltpu.make_async_remote_copy(
                src_ref=x_ref.at[pl.ds(offs_ref[r], MAX_ROWS)],
                dst_ref=stage_ref.at[dr],
                send_sem=send_sems.at[dr],
                recv_sem=recv_sems.at[dr],
                device_id=(r,),
                device_id_type=pl.DeviceIdType.MESH,
            )
            cnt = pltpu.make_async_remote_copy(
                src_ref=counts_ref,
                dst_ref=cstage_ref.at[dr],
                send_sem=csend_sems.at[dr],
                recv_sem=crecv_sems.at[dr],
                device_id=(r,),
                device_id_type=pl.DeviceIdType.MESH,
            )
            data.start()
            cnt.start()
            rdmas.append(data)
            rdmas.append(cnt)

        stage_ref[0] = x_ref[pl.ds(offs_ref[my], MAX_ROWS)]
        cstage_ref[0] = counts_ref[...]

        for rdma in rdmas:
            rdma.wait()

    stage, cstage = pl.pallas_call(
        body,
        out_shape=[
            jax.ShapeDtypeStruct((N_DEV, MAX_ROWS, D), jnp.float32),
            jax.ShapeDtypeStruct((N_DEV, 8, 128), jnp.int32),
        ],
        in_specs=[
            pl.BlockSpec(memory_space=pltpu.SMEM),
            pl.BlockSpec(memory_space=pltpu.VMEM),
            pl.BlockSpec(memory_space=pltpu.VMEM),
        ],
        out_specs=[
            pl.BlockSpec(memory_space=pltpu.VMEM),
            pl.BlockSpec(memory_space=pltpu.VMEM),
        ],
        scratch_shapes=[
            pltpu.SemaphoreType.DMA((N_DEV,)),
            pltpu.SemaphoreType.DMA((N_DEV,)),
            pltpu.SemaphoreType.DMA((N_DEV,)),
            pltpu.SemaphoreType.DMA((N_DEV,)),
        ],
        compiler_params=pltpu.CompilerParams(collective_id=0),
    )(offs, x_pad, counts_pad)

    me = lax.axis_index("i")
    perm = jnp.mod(me - jnp.arange(N_DEV), N_DEV)
    cmat = cstage[perm, 0, :N_DEV]
    cnt = jnp.take(cmat, me, axis=1)
    cum = jnp.cumsum(cnt)
    k = jnp.arange(S)
    s_k = jnp.searchsorted(cum, k, side="right")
    within = k - (cum[s_k] - cnt[s_k])
    flat = perm[s_k] * MAX_ROWS + within
    return stage.reshape(N_DEV * MAX_ROWS, D)[flat]


# baseline (device time: 338194 ns/iter reference)
import jax
import jax.numpy as jnp
from jax import lax
from jax.experimental import pallas as pl
from jax.experimental.pallas import tpu as pltpu

N_DEV = 4
MAX_ROWS = 576


def kernel(x, dest):
    return _impl(x, dest)[2]


def kernel_debug(x, dest):
    return _impl(x, dest)


def _impl(x, dest):
    S, D = x.shape

    order = jnp.argsort(dest, stable=True)
    counts = jnp.zeros((N_DEV,), jnp.int32).at[dest].add(1)
    offs = jnp.concatenate(
        [jnp.zeros((1,), jnp.int32), jnp.cumsum(counts)[:-1].astype(jnp.int32)]
    )
    dest_sorted = dest[order]
    slot_pos = dest_sorted * MAX_ROWS + (jnp.arange(S, dtype=jnp.int32) - offs[dest_sorted])
    x_send = (
        jnp.zeros((N_DEV * MAX_ROWS, D), jnp.float32)
        .at[slot_pos]
        .set(jnp.take(x, order, axis=0))
        .reshape(N_DEV, MAX_ROWS, D)
    )
    counts_pad = jnp.zeros((8, 128), jnp.int32).at[0, :N_DEV].set(counts)

    def body(x_ref, counts_ref, stage_ref, cstage_ref,
             send_sems, recv_sems, csend_sems, crecv_sems):
        my = lax.axis_index("i")

        barrier_sem = pltpu.get_barrier_semaphore()
        for dr in range(1, N_DEV):
            peer = lax.rem(my + dr, N_DEV)
            pl.semaphore_signal(
                barrier_sem, inc=1,
                device_id=(peer,), device_id_type=pl.DeviceIdType.MESH,
            )
        pl.semaphore_wait(barrier_sem, N_DEV - 1)

        rdmas = []
        for dr in range(1, N_DEV):
            r = lax.rem(my + dr, N_DEV)
            data = pltpu.make_async_remote_copy(
                src_ref=x_ref.at[r],
                dst_ref=stage_ref.at[dr],
                send_sem=send_sems.at[dr],
                recv_sem=recv_sems.at[dr],
                device_id=(r,),
                device_id_type=pl.DeviceIdType.MESH,
            )
            cnt = pltpu.make_async_remote_copy(
                src_ref=counts_ref,
                dst_ref=cstage_ref.at[dr],
                send_sem=csend_sems.at[dr],
                recv_sem=crecv_sems.at[dr],
                device_id=(r,),
                device_id_type=pl.DeviceIdType.MESH,
            )
            data.start()
            cnt.start()
            rdmas.append(data)
            rdmas.append(cnt)

        stage_ref[0] = x_ref[my]
        cstage_ref[0] = counts_ref[...]

        for rdma in rdmas:
            rdma.wait()

    stage, cstage = pl.pallas_call(
        body,
        out_shape=[
            jax.ShapeDtypeStruct((N_DEV, MAX_ROWS, D), jnp.float32),
            jax.ShapeDtypeStruct((N_DEV, 8, 128), jnp.int32),
        ],
        in_specs=[
            pl.BlockSpec(memory_space=pltpu.VMEM),
            pl.BlockSpec(memory_space=pltpu.VMEM),
        ],
        out_specs=[
            pl.BlockSpec(memory_space=pltpu.VMEM),
            pl.BlockSpec(memory_space=pltpu.VMEM),
        ],
        scratch_shapes=[
            pltpu.SemaphoreType.DMA((N_DEV,)),
            pltpu.SemaphoreType.DMA((N_DEV,)),
            pltpu.SemaphoreType.DMA((N_DEV,)),
            pltpu.SemaphoreType.DMA((N_DEV,)),
        ],
        compiler_params=pltpu.CompilerParams(collective_id=0),
    )(x_send, counts_pad)

    me = lax.axis_index("i")
    perm = jnp.mod(me - jnp.arange(N_DEV), N_DEV)
    cmat = cstage[perm, 0, :N_DEV]
    cnt = jnp.take(cmat, me, axis=1)
    cum = jnp.cumsum(cnt)
    k = jnp.arange(S)
    s_k = jnp.searchsorted(cum, k, side="right")
    within = k - (cum[s_k] - cnt[s_k])
    flat = perm[s_k] * MAX_ROWS + within
    return stage, cstage, stage.reshape(N_DEV * MAX_ROWS, D)[flat]


# device time: 98279 ns/iter; 3.4412x vs baseline; 3.4412x over previous
import jax
import jax.numpy as jnp
from jax import lax
from jax.experimental import pallas as pl
from jax.experimental.pallas import tpu as pltpu

N_DEV = 4
SUB = 8


def _counts_exchange(counts_pad):

    def body(c_ref, out_ref, send_sems, recv_sems):
        my = lax.axis_index("i")
        barrier_sem = pltpu.get_barrier_semaphore()
        for dr in range(1, N_DEV):
            peer = lax.rem(my + dr, N_DEV)
            pl.semaphore_signal(
                barrier_sem, inc=1,
                device_id=(peer,), device_id_type=pl.DeviceIdType.MESH,
            )
        pl.semaphore_wait(barrier_sem, N_DEV - 1)

        rdmas = []
        for dr in range(1, N_DEV):
            r = lax.rem(my + dr, N_DEV)
            rdma = pltpu.make_async_remote_copy(
                src_ref=c_ref,
                dst_ref=out_ref.at[dr],
                send_sem=send_sems.at[dr],
                recv_sem=recv_sems.at[dr],
                device_id=(r,),
                device_id_type=pl.DeviceIdType.MESH,
            )
            rdma.start()
            rdmas.append(rdma)
        out_ref[0] = c_ref[...]
        for rdma in rdmas:
            rdma.wait()

    return pl.pallas_call(
        body,
        out_shape=jax.ShapeDtypeStruct((N_DEV, 8, 128), jnp.int32),
        in_specs=[pl.BlockSpec(memory_space=pltpu.VMEM)],
        out_specs=pl.BlockSpec(memory_space=pltpu.VMEM),
        scratch_shapes=[
            pltpu.SemaphoreType.DMA((N_DEV,)),
            pltpu.SemaphoreType.DMA((N_DEV,)),
        ],
        compiler_params=pltpu.CompilerParams(collective_id=0),
    )(counts_pad)


def _scatter_rows(x8, dest, rowoff8, waits_recv, waits_send):
    S8 = x8.shape[0]
    S = S8 // SUB

    def body(dest_ref, rowoff_ref, wr_ref, ws_ref, x_ref, out_ref,
             send_sems, recv_sems, loc_sem):
        my = lax.axis_index("i")
        barrier_sem = pltpu.get_barrier_semaphore()
        for dr in range(1, N_DEV):
            peer = lax.rem(my + dr, N_DEV)
            pl.semaphore_signal(
                barrier_sem, inc=1,
                device_id=(peer,), device_id_type=pl.DeviceIdType.MESH,
            )
        pl.semaphore_wait(barrier_sem, N_DEV - 1)

        def send_one(j, carry):
            d = dest_ref[j]
            off8 = rowoff_ref[j]
            dr = lax.rem(d - my + N_DEV, N_DEV)

            @pl.when(dr != 0)
            def _():
                rdma = pltpu.make_async_remote_copy(
                    src_ref=x_ref.at[pl.ds(j * SUB, SUB)],
                    dst_ref=out_ref.at[pl.ds(off8, SUB)],
                    send_sem=send_sems.at[dr],
                    recv_sem=recv_sems.at[dr],
                    device_id=(d,),
                    device_id_type=pl.DeviceIdType.MESH,
                )
                rdma.start()

            @pl.when(dr == 0)
            def _():
                cp = pltpu.make_async_copy(
                    x_ref.at[pl.ds(j * SUB, SUB)],
                    out_ref.at[pl.ds(off8, SUB)],
                    loc_sem,
                )
                cp.start()

            return carry

        lax.fori_loop(0, S, send_one, 0)

        def wait_local(i, carry):
            pltpu.make_async_copy(
                x_ref.at[pl.ds(0, SUB)], out_ref.at[pl.ds(0, SUB)], loc_sem
            ).wait()
            return carry

        lax.fori_loop(0, wr_ref[0], wait_local, 0)

        for dr in range(1, N_DEV):
            dummy = pltpu.make_async_remote_copy(
                src_ref=x_ref.at[pl.ds(0, SUB)],
                dst_ref=out_ref.at[pl.ds(0, SUB)],
                send_sem=send_sems.at[dr],
                recv_sem=recv_sems.at[dr],
                device_id=(my,),
                device_id_type=pl.DeviceIdType.MESH,
            )

            def wait_send(i, carry):
                dummy.wait_send()
                return carry

            def wait_recv(i, carry):
                dummy.wait_recv()
                return carry

            lax.fori_loop(0, ws_ref[dr], wait_send, 0)
            lax.fori_loop(0, wr_ref[dr], wait_recv, 0)

    return pl.pallas_call(
        body,
        out_shape=jax.ShapeDtypeStruct((S8, 128), jnp.float32),
        in_specs=[
            pl.BlockSpec(memory_space=pltpu.SMEM),
            pl.BlockSpec(memory_space=pltpu.SMEM),
            pl.BlockSpec(memory_space=pltpu.SMEM),
            pl.BlockSpec(memory_space=pltpu.SMEM),
            pl.BlockSpec(memory_space=pltpu.VMEM),
        ],
        out_specs=pl.BlockSpec(memory_space=pltpu.VMEM),
        scratch_shapes=[
            pltpu.SemaphoreType.DMA((N_DEV,)),
            pltpu.SemaphoreType.DMA((N_DEV,)),
            pltpu.SemaphoreType.DMA,
        ],
        compiler_params=pltpu.CompilerParams(collective_id=1),
    )(dest, rowoff8, waits_recv, waits_send, x8)


def kernel(x, dest):
    S, D = x.shape
    dest = dest.astype(jnp.int32)

    oh = jax.nn.one_hot(dest, N_DEV, dtype=jnp.int32)
    counts = jnp.sum(oh, axis=0)
    cum = jnp.cumsum(oh, axis=0) - oh
    within = jnp.take_along_axis(cum, dest[:, None], axis=1)[:, 0]

    counts_pad = jnp.zeros((8, 128), jnp.int32).at[0, :N_DEV].set(counts)
    cstage = _counts_exchange(counts_pad)

    me = lax.axis_index("i")
    perm = jnp.mod(me - jnp.arange(N_DEV), N_DEV)
    C = cstage[perm, 0, :N_DEV]
    base = jnp.cumsum(C, axis=0) - C
    base_send = jnp.take(base, me, axis=0)
    rowoff = jnp.take(base_send, dest) + within
    rowoff8 = (rowoff * SUB).astype(jnp.int32)

    waits_recv = jnp.take(jnp.take(C, me, axis=1), perm)
    waits_send = jnp.take(
        jnp.take(C, me, axis=0), jnp.mod(me + jnp.arange(N_DEV), N_DEV)
    )

    x8 = x.reshape(S * SUB, 128)
    out8 = _scatter_rows(
        x8, dest, rowoff8,
        waits_recv.astype(jnp.int32), waits_send.astype(jnp.int32),
    )
    return out8.reshape(S, D)


# device time: 95559 ns/iter; 3.5391x vs baseline; 1.0285x over previous
import jax
import jax.numpy as jnp
from jax import lax
from jax.experimental import pallas as pl
from jax.experimental.pallas import tpu as pltpu

N_DEV = 4
SUB = 8


def _counts_exchange(counts_pad):

    def body(c_ref, out_ref, send_sems, recv_sems):
        my = lax.axis_index("i")
        barrier_sem = pltpu.get_barrier_semaphore()
        for dr in range(1, N_DEV):
            peer = lax.rem(my + dr, N_DEV)
            pl.semaphore_signal(
                barrier_sem, inc=1,
                device_id=(peer,), device_id_type=pl.DeviceIdType.MESH,
            )
        pl.semaphore_wait(barrier_sem, N_DEV - 1)

        rdmas = []
        for dr in range(1, N_DEV):
            r = lax.rem(my + dr, N_DEV)
            rdma = pltpu.make_async_remote_copy(
                src_ref=c_ref,
                dst_ref=out_ref.at[dr],
                send_sem=send_sems.at[dr],
                recv_sem=recv_sems.at[dr],
                device_id=(r,),
                device_id_type=pl.DeviceIdType.MESH,
            )
            rdma.start()
            rdmas.append(rdma)
        out_ref[0] = c_ref[...]
        for rdma in rdmas:
            rdma.wait()

    return pl.pallas_call(
        body,
        out_shape=jax.ShapeDtypeStruct((N_DEV, 8, 128), jnp.int32),
        in_specs=[pl.BlockSpec(memory_space=pltpu.VMEM)],
        out_specs=pl.BlockSpec(memory_space=pltpu.VMEM),
        scratch_shapes=[
            pltpu.SemaphoreType.DMA((N_DEV,)),
            pltpu.SemaphoreType.DMA((N_DEV,)),
        ],
        compiler_params=pltpu.CompilerParams(collective_id=0),
    )(counts_pad)


def _scatter_rows(x8, dest, rowoff8, waits_recv, waits_send):
    S8 = x8.shape[0]
    S = S8 // SUB

    def body(dest_ref, rowoff_ref, wr_ref, ws_ref, x_ref, out_ref,
             send_sems, recv_sems, loc_sem):
        my = lax.axis_index("i")
        barrier_sem = pltpu.get_barrier_semaphore()
        for dr in range(1, N_DEV):
            peer = lax.rem(my + dr, N_DEV)
            pl.semaphore_signal(
                barrier_sem, inc=1,
                device_id=(peer,), device_id_type=pl.DeviceIdType.MESH,
            )
        pl.semaphore_wait(barrier_sem, N_DEV - 1)

        def send_one(j, carry):
            d = dest_ref[j]
            off8 = rowoff_ref[j]
            dr = lax.rem(d - my + N_DEV, N_DEV)

            @pl.when(dr != 0)
            def _():
                rdma = pltpu.make_async_remote_copy(
                    src_ref=x_ref.at[pl.ds(j * SUB, SUB)],
                    dst_ref=out_ref.at[pl.ds(off8, SUB)],
                    send_sem=send_sems.at[dr],
                    recv_sem=recv_sems.at[dr],
                    device_id=(d,),
                    device_id_type=pl.DeviceIdType.MESH,
                )
                rdma.start()

            @pl.when(dr == 0)
            def _():
                cp = pltpu.make_async_copy(
                    x_ref.at[pl.ds(j * SUB, SUB)],
                    out_ref.at[pl.ds(off8, SUB)],
                    loc_sem,
                )
                cp.start()

            return carry

        lax.fori_loop(0, S, send_one, 0)

        def wait_local(i, carry):
            pltpu.make_async_copy(
                x_ref.at[pl.ds(0, SUB)], out_ref.at[pl.ds(0, SUB)], loc_sem
            ).wait()
            return carry

        lax.fori_loop(0, wr_ref[0], wait_local, 0)

        for dr in range(1, N_DEV):
            dummy = pltpu.make_async_remote_copy(
                src_ref=x_ref.at[pl.ds(0, SUB)],
                dst_ref=out_ref.at[pl.ds(0, SUB)],
                send_sem=send_sems.at[dr],
                recv_sem=recv_sems.at[dr],
                device_id=(my,),
                device_id_type=pl.DeviceIdType.MESH,
            )

            def wait_send(i, carry):
                dummy.wait_send()
                return carry

            def wait_recv(i, carry):
                dummy.wait_recv()
                return carry

            lax.fori_loop(0, ws_ref[dr], wait_send, 0)
            lax.fori_loop(0, wr_ref[dr], wait_recv, 0)

    return pl.pallas_call(
        body,
        out_shape=jax.ShapeDtypeStruct((S8, 128), jnp.float32),
        in_specs=[
            pl.BlockSpec(memory_space=pltpu.SMEM),
            pl.BlockSpec(memory_space=pltpu.SMEM),
            pl.BlockSpec(memory_space=pltpu.SMEM),
            pl.BlockSpec(memory_space=pltpu.SMEM),
            pl.BlockSpec(memory_space=pltpu.VMEM),
        ],
        out_specs=pl.BlockSpec(memory_space=pltpu.VMEM),
        scratch_shapes=[
            pltpu.SemaphoreType.DMA((N_DEV,)),
            pltpu.SemaphoreType.DMA((N_DEV,)),
            pltpu.SemaphoreType.DMA,
        ],
        compiler_params=pltpu.CompilerParams(collective_id=1),
    )(dest, rowoff8, waits_recv, waits_send, x8)


def kernel(x, dest):
    S, D = x.shape
    dest = dest.astype(jnp.int32)

    oh = jax.nn.one_hot(dest, N_DEV, dtype=jnp.int32)
    counts = jnp.sum(oh, axis=0)
    cum = jnp.cumsum(oh, axis=0) - oh
    within = jnp.sum(cum * oh, axis=1)

    counts_pad = jnp.zeros((8, 128), jnp.int32).at[0, :N_DEV].set(counts)
    cstage = _counts_exchange(counts_pad)

    me = lax.axis_index("i")
    perm = jnp.mod(me - jnp.arange(N_DEV), N_DEV)
    C = cstage[perm, 0, :N_DEV]
    base = jnp.cumsum(C, axis=0) - C
    base_send = jnp.take(base, me, axis=0)
    rowoff = jnp.sum(base_send[None, :] * oh, axis=1) + within
    rowoff8 = (rowoff * SUB).astype(jnp.int32)

    waits_recv = jnp.take(jnp.take(C, me, axis=1), perm)
    waits_send = jnp.take(
        jnp.take(C, me, axis=0), jnp.mod(me + jnp.arange(N_DEV), N_DEV)
    )

    x8 = x.reshape(S * SUB, 128)
    out8 = _scatter_rows(
        x8, dest, rowoff8,
        waits_recv.astype(jnp.int32), waits_send.astype(jnp.int32),
    )
    return out8.reshape(S, D)
